# baseline (speedup 1.0000x reference)
import functools

import jax
import jax.numpy as jnp
from jax import lax
from jax.experimental import pallas as pl
from jax.experimental.pallas import tpu as pltpu
from jax.experimental.pallas import tpu_sc as plsc

B = 16384
D = 128
NC = 2
NS = 16
NW = NC * NS
ROWS_PER_W = B // NW

BM = 4096


def _gather_body(user_table, item_table, uid, iid, ue_out, ie_out,
                 idx_u, idx_i, rows, sem):
    wid = lax.axis_index("s") * NC + lax.axis_index("c")
    rbase = wid * ROWS_PER_W
    pltpu.sync_copy(uid.at[pl.ds(rbase, ROWS_PER_W)], idx_u)
    pltpu.sync_copy(iid.at[pl.ds(rbase, ROWS_PER_W)], idx_i)
    pltpu.async_copy(user_table.at[idx_u], rows, sem).wait()
    pltpu.sync_copy(rows, ue_out.at[pl.ds(rbase, ROWS_PER_W)])
    pltpu.async_copy(item_table.at[idx_i], rows, sem).wait()
    pltpu.sync_copy(rows, ie_out.at[pl.ds(rbase, ROWS_PER_W)])


@functools.cache
def _sc_gather():
    return pl.kernel(
        _gather_body,
        out_type=(
            jax.ShapeDtypeStruct((B, D), jnp.float32),
            jax.ShapeDtypeStruct((B, D), jnp.float32),
        ),
        mesh=plsc.VectorSubcoreMesh(core_axis_name="c", subcore_axis_name="s"),
        scratch_types=[
            pltpu.VMEM((ROWS_PER_W,), jnp.int32),
            pltpu.VMEM((ROWS_PER_W,), jnp.int32),
            pltpu.VMEM((ROWS_PER_W, D), jnp.float32),
            pltpu.SemaphoreType.DMA,
        ],
    )


def _mlp_body(ue_ref, ie_ref, w1u_ref, w1i_ref, b1_ref, w2_ref, b2_ref,
              w3_ref, b3_ref, out_ref):
    h1 = jnp.dot(ue_ref[...], w1u_ref[...], preferred_element_type=jnp.float32)
    h1 += jnp.dot(ie_ref[...], w1i_ref[...], preferred_element_type=jnp.float32)
    h1 = jnp.maximum(h1 + b1_ref[...].reshape(1, 128), 0.0)
    h2 = jnp.dot(h1, w2_ref[...], preferred_element_type=jnp.float32)
    h2 = jnp.maximum(h2 + b2_ref[...].reshape(1, 64), 0.0)
    r = lax.dot_general(w3_ref[...], h2, (((0,), (1,)), ((), ())),
                        preferred_element_type=jnp.float32) + b3_ref[0]
    out_ref[...] = r.reshape(1, 1, r.shape[-1])


def _mlp(ue, ie, W1a, W1b, b1, W2, b2, W3, b3):
    return pl.pallas_call(
        _mlp_body,
        grid=(B // BM,),
        in_specs=[
            pl.BlockSpec((BM, D), lambda i: (i, 0)),
            pl.BlockSpec((BM, D), lambda i: (i, 0)),
            pl.BlockSpec((D, 128), lambda i: (0, 0)),
            pl.BlockSpec((D, 128), lambda i: (1, 0)),
            pl.BlockSpec((128,), lambda i: (0,)),
            pl.BlockSpec((128, 64), lambda i: (0, 0)),
            pl.BlockSpec((64,), lambda i: (0,)),
            pl.BlockSpec((64, 1), lambda i: (0, 0)),
            pl.BlockSpec((1,), lambda i: (0,)),
        ],
        out_specs=pl.BlockSpec((1, 1, BM), lambda i: (i, 0, 0)),
        out_shape=jax.ShapeDtypeStruct((B // BM, 1, BM), jnp.float32),
    )(ue, ie, W1a, W1b, b1, W2, b2, W3, b3)


def kernel(user_id, item_id, user_table, item_table, W1, b1, W2, b2, W3, b3):
    ue, ie = _sc_gather()(user_table, item_table, user_id, item_id)
    out = _mlp(ue, ie, W1, W1, b1, W2, b2, W3, b3)
    return out.reshape(B)


# speedup vs baseline: 1.0057x; 1.0057x over previous
import functools

import jax
import jax.numpy as jnp
from jax import lax
from jax.experimental import pallas as pl
from jax.experimental.pallas import tpu as pltpu
from jax.experimental.pallas import tpu_sc as plsc

B = 16384
D = 128
NC = 2
NS = 16
NW = NC * NS
ROWS_PER_W = B // NW

BM = 4096


def _gather_body(user_table, item_table, uid, iid, x_out,
                 idx_u, idx_i, rows, sem):
    wid = lax.axis_index("s") * NC + lax.axis_index("c")
    rbase = wid * ROWS_PER_W
    pltpu.sync_copy(uid.at[pl.ds(rbase, ROWS_PER_W)], idx_u)
    pltpu.sync_copy(iid.at[pl.ds(rbase, ROWS_PER_W)], idx_i)
    pltpu.async_copy(user_table.at[idx_u], rows, sem).wait()
    pltpu.sync_copy(rows, x_out.at[pl.ds(rbase, ROWS_PER_W), pl.ds(0, D)])
    pltpu.async_copy(item_table.at[idx_i], rows, sem).wait()
    pltpu.sync_copy(rows, x_out.at[pl.ds(rbase, ROWS_PER_W), pl.ds(D, D)])


@functools.cache
def _sc_gather():
    return pl.kernel(
        _gather_body,
        out_type=jax.ShapeDtypeStruct((B, 2 * D), jnp.float32),
        mesh=plsc.VectorSubcoreMesh(core_axis_name="c", subcore_axis_name="s"),
        scratch_types=[
            pltpu.VMEM((ROWS_PER_W,), jnp.int32),
            pltpu.VMEM((ROWS_PER_W,), jnp.int32),
            pltpu.VMEM((ROWS_PER_W, D), jnp.float32),
            pltpu.SemaphoreType.DMA,
        ],
    )


def _mlp_body(x_ref, w1_ref, b1_ref, w2_ref, b2_ref,
              w3_ref, b3_ref, out_ref):
    h1 = jnp.dot(x_ref[...], w1_ref[...], preferred_element_type=jnp.float32)
    h1 = jnp.maximum(h1 + b1_ref[...].reshape(1, 128), 0.0)
    h2 = jnp.dot(h1, w2_ref[...], preferred_element_type=jnp.float32)
    h2 = jnp.maximum(h2 + b2_ref[...].reshape(1, 64), 0.0)
    r = lax.dot_general(w3_ref[...], h2, (((0,), (1,)), ((), ())),
                        preferred_element_type=jnp.float32) + b3_ref[0]
    out_ref[...] = r.reshape(1, 1, r.shape[-1])


def _mlp(x, W1, b1, W2, b2, W3, b3):
    return pl.pallas_call(
        _mlp_body,
        grid=(B // BM,),
        in_specs=[
            pl.BlockSpec((BM, 2 * D), lambda i: (i, 0)),
            pl.BlockSpec((2 * D, 128), lambda i: (0, 0)),
            pl.BlockSpec((128,), lambda i: (0,)),
            pl.BlockSpec((128, 64), lambda i: (0, 0)),
            pl.BlockSpec((64,), lambda i: (0,)),
            pl.BlockSpec((64, 1), lambda i: (0, 0)),
            pl.BlockSpec((1,), lambda i: (0,)),
        ],
        out_specs=pl.BlockSpec((1, 1, BM), lambda i: (i, 0, 0)),
        out_shape=jax.ShapeDtypeStruct((B // BM, 1, BM), jnp.float32),
    )(x, W1, b1, W2, b2, W3, b3)


def kernel(user_id, item_id, user_table, item_table, W1, b1, W2, b2, W3, b3):
    x = _sc_gather()(user_table, item_table, user_id, item_id)
    out = _mlp(x, W1, b1, W2, b2, W3, b3)
    return out.reshape(B)
